# baseline (speedup 1.0000x reference)
import functools

import jax
import jax.numpy as jnp
from jax import lax
from jax.experimental import pallas as pl
from jax.experimental.pallas import tpu as pltpu
from jax.experimental.pallas import tpu_sc as plsc

_K = 4
_LANES = 16
_IDXW = 128


@functools.lru_cache(maxsize=None)
def _build(n, m, nc, ns, chunk):
    nw = nc * ns
    ppw = n // nw
    nchunk = ppw // chunk
    ngrp = chunk // _LANES
    nsub = chunk // _IDXW
    inv_m = 1.0 / float(m)

    mesh = plsc.VectorSubcoreMesh(core_axis_name="c", subcore_axis_name="s")

    @functools.partial(
        pl.kernel,
        out_type=jax.ShapeDtypeStruct((n,), jnp.float32),
        mesh=mesh,
        compiler_params=pltpu.CompilerParams(
            needs_layout_passes=False, use_tc_tiling_on_sc=False
        ),
        scratch_types=[
            pltpu.VMEM((m,), jnp.float32),
            pltpu.VMEM_SHARED((m,), jnp.float32),
            pltpu.VMEM_SHARED((m,), jnp.float32),
            pltpu.VMEM_SHARED((m,), jnp.float32),
            pltpu.VMEM((chunk,), jnp.float32),
            pltpu.VMEM((chunk,), jnp.float32),
            pltpu.VMEM((chunk,), jnp.int32),
            pltpu.VMEM((chunk,), jnp.float32),
            pltpu.VMEM((chunk,), jnp.float32),
            pltpu.VMEM((chunk,), jnp.float32),
            pltpu.VMEM((chunk,), jnp.float32),
            pltpu.SemaphoreType.DMA,
        ],
    )
    def ppoly_kernel(
        x_hbm, c_hbm, out_hbm,
        tloc, t1, t2, t3, xv, tv, idxv, g1, g2, g3, yv, sem,
    ):
        cid = lax.axis_index("c")
        sid = lax.axis_index("s")
        wid = sid * nc + cid

        @pl.when(sid == 0)
        def _stage():
            pltpu.sync_copy(c_hbm.at[1], t1)
            pltpu.sync_copy(c_hbm.at[2], t2)
            pltpu.sync_copy(c_hbm.at[3], t3)

        pltpu.sync_copy(c_hbm.at[0], tloc)
        plsc.subcore_barrier()

        def chunk_body(g, carry):
            off = pl.multiple_of(wid * ppw + g * chunk, _IDXW)
            pltpu.sync_copy(x_hbm.at[pl.ds(off, chunk)], xv)

            def p1(i, carry1):
                xx = xv[pl.ds(i * _LANES, _LANES)]
                ii = (xx * float(m)).astype(jnp.int32)
                ii = jnp.clip(ii, 0, m - 1)
                idxv[pl.ds(i * _LANES, _LANES)] = ii
                tv[pl.ds(i * _LANES, _LANES)] = xx - ii.astype(jnp.float32) * inv_m
                return carry1

            lax.fori_loop(0, ngrp, p1, 0, unroll=2)

            copies = []
            for j in range(nsub):
                sl = pl.ds(j * _IDXW, _IDXW)
                ids = idxv.at[sl]
                copies.append(pltpu.async_copy(t1.at[ids], g1.at[sl], sem))
                copies.append(pltpu.async_copy(t2.at[ids], g2.at[sl], sem))
                copies.append(pltpu.async_copy(t3.at[ids], g3.at[sl], sem))
            for d in copies:
                d.wait()

            def p2(i, carry2):
                sl = pl.ds(i * _LANES, _LANES)
                tt = tv[sl]
                r0 = plsc.load_gather(tloc, [idxv[sl]])
                yv[sl] = ((r0 * tt + g1[sl]) * tt + g2[sl]) * tt + g3[sl]
                return carry2

            lax.fori_loop(0, ngrp, p2, 0, unroll=2)

            pltpu.sync_copy(yv, out_hbm.at[pl.ds(off, chunk)])
            return carry

        lax.fori_loop(0, nchunk, chunk_body, 0)

    return ppoly_kernel


def kernel(x_eval, c, breaks):
    del breaks
    n = x_eval.shape[0]
    m = c.shape[1]
    return _build(n, m, 2, 16, 4096)(x_eval, c)


# speedup vs baseline: 1.0390x; 1.0390x over previous
import functools

import jax
import jax.numpy as jnp
from jax import lax
from jax.experimental import pallas as pl
from jax.experimental.pallas import tpu as pltpu
from jax.experimental.pallas import tpu_sc as plsc

_K = 4
_LANES = 16
_IDXW = 128


@functools.lru_cache(maxsize=None)
def _build(n, m, nc, ns, chunk):
    nw = nc * ns
    ppw = n // nw
    nchunk = ppw // chunk
    ngrp = chunk // _LANES
    nsub = chunk // _IDXW
    inv_m = 1.0 / float(m)

    mesh = plsc.VectorSubcoreMesh(core_axis_name="c", subcore_axis_name="s")

    @functools.partial(
        pl.kernel,
        out_type=jax.ShapeDtypeStruct((n,), jnp.float32),
        mesh=mesh,
        compiler_params=pltpu.CompilerParams(
            needs_layout_passes=False, use_tc_tiling_on_sc=False
        ),
        scratch_types=[
            pltpu.VMEM_SHARED((m,), jnp.float32),
            pltpu.VMEM_SHARED((m,), jnp.float32),
            pltpu.VMEM_SHARED((m,), jnp.float32),
            pltpu.VMEM_SHARED((m,), jnp.float32),
            pltpu.VMEM((chunk,), jnp.float32),
            pltpu.VMEM((chunk,), jnp.int32),
            pltpu.VMEM((chunk,), jnp.float32),
            pltpu.VMEM((chunk,), jnp.float32),
            pltpu.VMEM((chunk,), jnp.float32),
            pltpu.VMEM((chunk,), jnp.float32),
            pltpu.VMEM((chunk,), jnp.float32),
            pltpu.SemaphoreType.DMA,
        ],
    )
    def ppoly_kernel(
        x_hbm, c_hbm, out_hbm,
        t0, t1, t2, t3, xv, idxv, g0, g1, g2, g3, yv, sem,
    ):
        cid = lax.axis_index("c")
        sid = lax.axis_index("s")
        wid = sid * nc + cid

        @pl.when(sid == 0)
        def _stage():
            pltpu.sync_copy(c_hbm.at[0], t0)
            pltpu.sync_copy(c_hbm.at[1], t1)
            pltpu.sync_copy(c_hbm.at[2], t2)
            pltpu.sync_copy(c_hbm.at[3], t3)

        plsc.subcore_barrier()

        def chunk_body(g, carry):
            off = pl.multiple_of(wid * ppw + g * chunk, _IDXW)
            pltpu.sync_copy(x_hbm.at[pl.ds(off, chunk)], xv)

            def p1(i, carry1):
                xx = xv[pl.ds(i * _LANES, _LANES)]
                ii = (xx * float(m)).astype(jnp.int32)
                ii = jnp.clip(ii, 0, m - 1)
                idxv[pl.ds(i * _LANES, _LANES)] = ii
                return carry1

            lax.fori_loop(0, ngrp, p1, 0, unroll=2)

            copies = []
            for j in range(nsub):
                sl = pl.ds(j * _IDXW, _IDXW)
                ids = idxv.at[sl]
                copies.append(pltpu.async_copy(t0.at[ids], g0.at[sl], sem))
                copies.append(pltpu.async_copy(t1.at[ids], g1.at[sl], sem))
                copies.append(pltpu.async_copy(t2.at[ids], g2.at[sl], sem))
                copies.append(pltpu.async_copy(t3.at[ids], g3.at[sl], sem))
            for d in copies:
                d.wait()

            def p2(i, carry2):
                sl = pl.ds(i * _LANES, _LANES)
                tt = xv[sl] - idxv[sl].astype(jnp.float32) * inv_m
                yv[sl] = ((g0[sl] * tt + g1[sl]) * tt + g2[sl]) * tt + g3[sl]
                return carry2

            lax.fori_loop(0, ngrp, p2, 0, unroll=2)

            pltpu.sync_copy(yv, out_hbm.at[pl.ds(off, chunk)])
            return carry

        lax.fori_loop(0, nchunk, chunk_body, 0)

    return ppoly_kernel


def kernel(x_eval, c, breaks):
    del breaks
    n = x_eval.shape[0]
    m = c.shape[1]
    return _build(n, m, 2, 16, 16384)(x_eval, c)


# speedup vs baseline: 1.0679x; 1.0278x over previous
import functools

import jax
import jax.numpy as jnp
from jax import lax
from jax.experimental import pallas as pl
from jax.experimental.pallas import tpu as pltpu
from jax.experimental.pallas import tpu_sc as plsc

_K = 4
_LANES = 16
_IDXW = 128


@functools.lru_cache(maxsize=None)
def _build(n, m, nc, ns, chunk):
    nw = nc * ns
    ppw = n // nw
    nchunk = ppw // chunk
    ngrp = chunk // _LANES
    nsub = chunk // _IDXW
    inv_m = 1.0 / float(m)

    mesh = plsc.VectorSubcoreMesh(core_axis_name="c", subcore_axis_name="s")

    @functools.partial(
        pl.kernel,
        out_type=jax.ShapeDtypeStruct((n,), jnp.float32),
        mesh=mesh,
        compiler_params=pltpu.CompilerParams(
            needs_layout_passes=False, use_tc_tiling_on_sc=False
        ),
        scratch_types=[
            pltpu.VMEM_SHARED((m,), jnp.float32),
            pltpu.VMEM_SHARED((m,), jnp.float32),
            pltpu.VMEM_SHARED((m,), jnp.float32),
            pltpu.VMEM_SHARED((m,), jnp.float32),
            pltpu.VMEM((chunk,), jnp.float32),
            pltpu.VMEM((chunk,), jnp.int32),
            pltpu.VMEM((chunk,), jnp.float32),
            pltpu.VMEM((chunk,), jnp.float32),
            pltpu.VMEM((chunk,), jnp.float32),
            pltpu.VMEM((chunk,), jnp.float32),
            pltpu.VMEM((chunk,), jnp.float32),
            pltpu.SemaphoreType.DMA,
        ],
    )
    def ppoly_kernel(
        x_hbm, c_hbm, out_hbm,
        t0, t1, t2, t3, xv, idxv, g0, g1, g2, g3, yv, sem,
    ):
        cid = lax.axis_index("c")
        sid = lax.axis_index("s")
        wid = sid * nc + cid

        @pl.when(sid == 0)
        def _stage():
            pltpu.sync_copy(c_hbm.at[0], t0)
            pltpu.sync_copy(c_hbm.at[1], t1)
            pltpu.sync_copy(c_hbm.at[2], t2)
            pltpu.sync_copy(c_hbm.at[3], t3)

        plsc.subcore_barrier()

        def chunk_body(g, carry):
            off = pl.multiple_of(wid * ppw + g * chunk, _IDXW)
            pltpu.sync_copy(x_hbm.at[pl.ds(off, chunk)], xv)

            def p1(i, carry1):
                xx = xv[pl.ds(i * _LANES, _LANES)]
                ii = (xx * float(m)).astype(jnp.int32)
                ii = jnp.clip(ii, 0, m - 1)
                idxv[pl.ds(i * _LANES, _LANES)] = ii
                return carry1

            lax.fori_loop(0, ngrp, p1, 0, unroll=2)

            copies = []
            for j in range(nsub):
                sl = pl.ds(j * _IDXW, _IDXW)
                ids = idxv.at[sl]
                copies.append(pltpu.async_copy(t0.at[ids], g0.at[sl], sem))
                copies.append(pltpu.async_copy(t1.at[ids], g1.at[sl], sem))
                copies.append(pltpu.async_copy(t2.at[ids], g2.at[sl], sem))
                copies.append(pltpu.async_copy(t3.at[ids], g3.at[sl], sem))
            for d in copies:
                d.wait()

            def p2(i, carry2):
                sl = pl.ds(i * _LANES, _LANES)
                tt = xv[sl] - idxv[sl].astype(jnp.float32) * inv_m
                yv[sl] = ((g0[sl] * tt + g1[sl]) * tt + g2[sl]) * tt + g3[sl]
                return carry2

            lax.fori_loop(0, ngrp, p2, 0, unroll=2)

            pltpu.sync_copy(yv, out_hbm.at[pl.ds(off, chunk)])
            return carry

        lax.fori_loop(0, nchunk, chunk_body, 0)

    return ppoly_kernel


def kernel(x_eval, c, breaks):
    del breaks
    n = x_eval.shape[0]
    m = c.shape[1]
    return _build(n, m, 2, 16, 8192)(x_eval, c)
